# baseline (speedup 1.0000x reference)
import functools

import jax
import jax.numpy as jnp
import numpy as np
from jax import lax
from jax.experimental import pallas as pl
from jax.experimental.pallas import tpu as pltpu
from jax.experimental.pallas import tpu_sc as plsc

_VOCAB = 100000
_D = 1024
_B = 4
_L = 4096

_NC = 2
_NS = 16
_NW = _NC * _NS
_LPW = _L // _NW
_C = 16
_NCHUNK = _LPW // _C


def _pe_table() -> np.ndarray:
    pos = np.arange(_L, dtype=np.float32)[:, None]
    dim = np.arange(_D, dtype=np.float32)
    pe = np.zeros((_L, _D), dtype=np.float32)
    pe[:, 0::2] = np.sin(pos / 10000.0 ** (dim[0::2] / _D)).astype(np.float32)
    pe[:, 1::2] = np.cos(pos / 10000.0 ** (dim[1::2] / _D)).astype(np.float32)
    return pe


_PE = _pe_table()


def _body(x_ref, pe_ref, table_ref, out_ref,
          idx_v, pe_v, r0, r1, r2, r3,
          gs0, gs1, gs2, gs3, os0, os1, os2, os3, ps0):
    rows = (r0, r1, r2, r3)
    gsem = (gs0, gs1, gs2, gs3)
    osem = (os0, os1, os2, os3)

    wid = lax.axis_index("s") * _NC + lax.axis_index("c")
    l0 = wid * _LPW

    for b in range(_B):
        pltpu.sync_copy(x_ref.at[pl.ds(b * _L + l0, _LPW)], idx_v.at[b])

    def issue_pe(c):
        pltpu.async_copy(pe_ref.at[pl.ds(l0 + c * _C, _C)], pe_v, ps0)

    def issue_gather(c, b):
        idx = idx_v.at[b, pl.ds(c * _C, _C)]
        pltpu.async_copy(table_ref.at[idx], rows[b], gsem[b])

    def issue_scatter(c, b):
        row0 = b * _L + l0 + c * _C
        pltpu.async_copy(rows[b], out_ref.at[pl.ds(row0, _C)], osem[b])

    def wait_pe():
        pltpu.make_async_copy(pe_ref.at[pl.ds(0, _C)], pe_v, ps0).wait()

    def wait_gather(b):
        pltpu.make_async_copy(table_ref.at[pl.ds(0, _C)], rows[b],
                              gsem[b]).wait()

    def wait_scatter(b):
        pltpu.make_async_copy(table_ref.at[pl.ds(0, _C)], rows[b],
                              osem[b]).wait()

    def add_pe(rv):
        @pl.loop(0, _C)
        def _row(r):
            rr = rv.at[r]
            pr = pe_v.at[r]

            @pl.loop(0, _D // 16, unroll=16)
            def _grp(j):
                sl = pl.ds(j * 16, 16)
                plsc.addupdate(rr.at[sl], pr[sl])

    issue_pe(0)
    for b in range(_B):
        issue_gather(0, b)

    last = _NCHUNK - 1

    @pl.loop(0, _NCHUNK)
    def _chunk(c):
        cn = jnp.minimum(c + 1, last)
        wait_pe()
        for b in range(_B):
            wait_gather(b)
            add_pe(rows[b])
            issue_scatter(c, b)
        issue_pe(cn)
        for b in range(_B):
            wait_scatter(b)
            issue_gather(cn, b)

    wait_pe()
    for b in range(_B):
        wait_gather(b)


@functools.lru_cache(maxsize=1)
def _build():
    return pl.kernel(
        _body,
        out_type=jax.ShapeDtypeStruct((_B * _L, _D), jnp.float32),
        mesh=plsc.VectorSubcoreMesh(core_axis_name="c", subcore_axis_name="s",
                                    num_cores=_NC, num_subcores=_NS),
        scratch_types=(
            [pltpu.VMEM((_B, _LPW), jnp.int32),
             pltpu.VMEM((_C, _D), jnp.float32)]
            + [pltpu.VMEM((_C, _D), jnp.float32)
               for _ in range(_B)]
            + [pltpu.SemaphoreType.DMA for _ in range(2 * _B + 1)]
        ),
    )


def kernel(x, table):
    pe = jnp.asarray(_PE)
    x_flat = x.reshape(_B * _L).astype(jnp.int32)
    out = _build()(x_flat, pe, table)
    return out.reshape(_B, _L, _D)
